# baseline (speedup 1.0000x reference)
import functools

import jax
import jax.numpy as jnp
from jax import lax
from jax.experimental import pallas as pl
from jax.experimental.pallas import tpu as pltpu
from jax.experimental.pallas import tpu_sc as plsc

_BLK = 2048


def _concat_body(x_ref, pe_ref, out_ref):
    d = x_ref.shape[2]
    out_ref[0, :, :d] = x_ref[0]
    out_ref[0, :, d:] = pe_ref[...]


def _tc_concat(x, x_pos):
    B, L, D = x.shape
    P = x_pos.shape[1]
    grid = (B, L // _BLK)
    return pl.pallas_call(
        _concat_body,
        grid=grid,
        in_specs=[
            pl.BlockSpec((1, _BLK, D), lambda b, i: (b, i, 0)),
            pl.BlockSpec((_BLK, P), lambda b, i: (i, 0)),
        ],
        out_specs=pl.BlockSpec((1, _BLK, D + P), lambda b, i: (b, i, 0)),
        out_shape=jax.ShapeDtypeStruct((B, L, D + P), x.dtype),
        compiler_params=pltpu.CompilerParams(
            dimension_semantics=("parallel", "parallel"),
        ),
    )(x, x_pos)


def _sc_gather(pe_weight, pos):
    V, P = pe_weight.shape
    L = pos.shape[0]
    info = plsc.get_sparse_core_info()
    nw = info.num_cores * info.num_subcores
    rows_per_w = L // nw
    mesh = plsc.VectorSubcoreMesh(core_axis_name="c", subcore_axis_name="s")

    @functools.partial(
        pl.kernel,
        mesh=mesh,
        out_type=jax.ShapeDtypeStruct((L, P), pe_weight.dtype),
        scratch_types=[
            pltpu.VMEM((rows_per_w,), jnp.int32),
            pltpu.VMEM((rows_per_w, P), pe_weight.dtype),
            pltpu.SemaphoreType.DMA,
        ],
    )
    def gather_k(pe_hbm, pos_hbm, out_hbm, idx_v, rows_v, sem):
        wid = lax.axis_index("s") * info.num_cores + lax.axis_index("c")
        base = wid * rows_per_w
        pltpu.sync_copy(pos_hbm.at[pl.ds(base, rows_per_w)], idx_v)
        pltpu.async_copy(pe_hbm.at[idx_v], rows_v, sem).wait()
        pltpu.sync_copy(rows_v, out_hbm.at[pl.ds(base, rows_per_w)])

    return gather_k(pe_weight, pos)


def kernel(x, pe_weight, pos):
    x_pos = _sc_gather(pe_weight, pos)
    return _tc_concat(x, x_pos)


# speedup vs baseline: 1.0269x; 1.0269x over previous
import functools

import jax
import jax.numpy as jnp
from jax import lax
from jax.experimental import pallas as pl
from jax.experimental.pallas import tpu as pltpu
from jax.experimental.pallas import tpu_sc as plsc

_BLK = 2048


def _concat_body(x_ref, pe_ref, out_ref):
    d = x_ref.shape[2]
    out_ref[0, :, :d] = x_ref[0]
    out_ref[0, :, d:] = pe_ref[...]


def _tc_concat(x, x_pos):
    B, L, D = x.shape
    P = x_pos.shape[1]
    grid = (L // _BLK, B)
    return pl.pallas_call(
        _concat_body,
        grid=grid,
        in_specs=[
            pl.BlockSpec((1, _BLK, D), lambda i, b: (b, i, 0)),
            pl.BlockSpec((_BLK, P), lambda i, b: (i, 0)),
        ],
        out_specs=pl.BlockSpec((1, _BLK, D + P), lambda i, b: (b, i, 0)),
        out_shape=jax.ShapeDtypeStruct((B, L, D + P), x.dtype),
        compiler_params=pltpu.CompilerParams(
            dimension_semantics=("parallel", "parallel"),
        ),
    )(x, x_pos)


def _sc_gather(pe_weight, pos):
    V, P = pe_weight.shape
    L = pos.shape[0]
    info = plsc.get_sparse_core_info()
    nw = info.num_cores * info.num_subcores
    rows_per_w = L // nw
    mesh = plsc.VectorSubcoreMesh(core_axis_name="c", subcore_axis_name="s")

    @functools.partial(
        pl.kernel,
        mesh=mesh,
        out_type=jax.ShapeDtypeStruct((L, P), pe_weight.dtype),
        scratch_types=[
            pltpu.VMEM((rows_per_w,), jnp.int32),
            pltpu.VMEM((rows_per_w, P), pe_weight.dtype),
            pltpu.SemaphoreType.DMA,
        ],
    )
    def gather_k(pe_hbm, pos_hbm, out_hbm, idx_v, rows_v, sem):
        wid = lax.axis_index("s") * info.num_cores + lax.axis_index("c")
        base = wid * rows_per_w
        pltpu.sync_copy(pos_hbm.at[pl.ds(base, rows_per_w)], idx_v)
        pltpu.async_copy(pe_hbm.at[idx_v], rows_v, sem).wait()
        pltpu.sync_copy(rows_v, out_hbm.at[pl.ds(base, rows_per_w)])

    return gather_k(pe_weight, pos)


def kernel(x, pe_weight, pos):
    x_pos = _sc_gather(pe_weight, pos)
    return _tc_concat(x, x_pos)


# speedup vs baseline: 1.0351x; 1.0081x over previous
import functools

import jax
import jax.numpy as jnp
from jax import lax
from jax.experimental import pallas as pl
from jax.experimental.pallas import tpu as pltpu
from jax.experimental.pallas import tpu_sc as plsc

_BLK = 2048


def _concat_body(x_ref, pe_ref, out_ref):
    d = x_ref.shape[2]
    out_ref[0, :, :d] = x_ref[0]
    out_ref[0, :, d:] = pe_ref[...]


def _tc_concat(x, x_pos):
    B, L, D = x.shape
    P = x_pos.shape[1]
    grid = (L // _BLK, B)
    return pl.pallas_call(
        _concat_body,
        grid=grid,
        in_specs=[
            pl.BlockSpec((1, _BLK, D), lambda i, b: (b, i, 0)),
            pl.BlockSpec((_BLK, P), lambda i, b: (i, 0)),
        ],
        out_specs=pl.BlockSpec((1, _BLK, D + P), lambda i, b: (b, i, 0)),
        out_shape=jax.ShapeDtypeStruct((B, L, D + P), x.dtype),
        compiler_params=pltpu.CompilerParams(
            dimension_semantics=("parallel", "parallel"),
        ),
    )(x, x_pos)


def _sc_gather(pe_weight, pos):
    V, P = pe_weight.shape
    L = pos.shape[0]
    info = plsc.get_sparse_core_info()
    num_cores = 1
    nw = num_cores * info.num_subcores
    rows_per_w = L // nw
    mesh = plsc.VectorSubcoreMesh(
        core_axis_name="c", subcore_axis_name="s", num_cores=num_cores
    )

    @functools.partial(
        pl.kernel,
        mesh=mesh,
        out_type=jax.ShapeDtypeStruct((L, P), pe_weight.dtype),
        scratch_types=[
            pltpu.VMEM((rows_per_w,), jnp.int32),
            pltpu.VMEM((rows_per_w, P), pe_weight.dtype),
            pltpu.SemaphoreType.DMA,
        ],
    )
    def gather_k(pe_hbm, pos_hbm, out_hbm, idx_v, rows_v, sem):
        wid = lax.axis_index("s") * num_cores + lax.axis_index("c")
        base = wid * rows_per_w
        pltpu.sync_copy(pos_hbm.at[pl.ds(base, rows_per_w)], idx_v)
        pltpu.async_copy(pe_hbm.at[idx_v], rows_v, sem).wait()
        pltpu.sync_copy(rows_v, out_hbm.at[pl.ds(base, rows_per_w)])

    return gather_k(pe_weight, pos)


def kernel(x, pe_weight, pos):
    x_pos = _sc_gather(pe_weight, pos)
    return _tc_concat(x, x_pos)
